# baseline (speedup 1.0000x reference)
import functools

import jax
import jax.numpy as jnp
import numpy as np
from jax import lax
from jax.experimental import pallas as pl
from jax.experimental.pallas import tpu as pltpu
from jax.experimental.pallas import tpu_sc as plsc

H = 16
OMEGA = 30.0
CH = 128
NW = 32


def _gather_kernel(epad):
    rows = epad // NW // CH
    mesh = plsc.VectorSubcoreMesh(core_axis_name="c", subcore_axis_name="s")

    @functools.partial(
        pl.kernel,
        mesh=mesh,
        out_type=(
            jax.ShapeDtypeStruct((2, epad, H), jnp.float32),
            jax.ShapeDtypeStruct((2, epad), jnp.float32),
        ),
        scratch_types=[
            pltpu.VMEM((CH,), jnp.int32),
            pltpu.VMEM((CH,), jnp.int32),
            pltpu.VMEM((CH, H), jnp.float32),
            pltpu.VMEM((CH, H), jnp.float32),
            pltpu.VMEM((CH,), jnp.float32),
            pltpu.VMEM((CH,), jnp.float32),
            pltpu.VMEM((CH,), jnp.float32),
            pltpu.SemaphoreType.DMA,
        ],
        compiler_params=pltpu.CompilerParams(use_tc_tiling_on_sc=False),
    )
    def gather(ip_hbm, up_hbm, iemb_hbm, uemb_hbm, it_hbm, ut_hbm, et_hbm,
               emb_out, rel_out,
               ipv, upv, iembv, uembv, itv, utv, etv, sem):
        wid = lax.axis_index("s") * 2 + lax.axis_index("c")
        base = wid * (epad // NW)

        def body(r, carry):
            off = base + r * CH
            pltpu.sync_copy(ip_hbm.at[pl.ds(off, CH)], ipv)
            pltpu.sync_copy(up_hbm.at[pl.ds(off, CH)], upv)
            pltpu.sync_copy(et_hbm.at[pl.ds(off, CH)], etv)
            c1 = pltpu.async_copy(iemb_hbm.at[ipv], iembv, sem)
            c2 = pltpu.async_copy(uemb_hbm.at[upv], uembv, sem)
            c3 = pltpu.async_copy(it_hbm.at[ipv], itv, sem)
            c4 = pltpu.async_copy(ut_hbm.at[upv], utv, sem)
            c1.wait()
            c2.wait()
            c3.wait()
            c4.wait()
            for k in range(CH // 16):
                sl = pl.ds(k * 16, 16)
                itv[sl] = itv[sl] - etv[sl]
                utv[sl] = utv[sl] - etv[sl]
            pltpu.sync_copy(iembv, emb_out.at[0, pl.ds(off, CH)])
            pltpu.sync_copy(uembv, emb_out.at[1, pl.ds(off, CH)])
            pltpu.sync_copy(itv, rel_out.at[0, pl.ds(off, CH)])
            pltpu.sync_copy(utv, rel_out.at[1, pl.ds(off, CH)])
            return carry

        lax.fori_loop(0, rows, body, 0)

    return gather


def _scatter_kernel(epad, n_u, n_i, acc_rows):
    rows = epad // 16 // CH
    zr = acc_rows // 16
    mesh = plsc.VectorSubcoreMesh(core_axis_name="c", subcore_axis_name="s")

    @functools.partial(
        pl.kernel,
        mesh=mesh,
        out_type=(
            jax.ShapeDtypeStruct((n_u, H), jnp.float32),
            jax.ShapeDtypeStruct((n_i, H), jnp.float32),
        ),
        scratch_types=[
            pltpu.VMEM((CH,), jnp.int32),
            pltpu.VMEM((CH, H), jnp.float32),
            pltpu.VMEM_SHARED((acc_rows, H), jnp.float32),
        ],
        compiler_params=pltpu.CompilerParams(use_tc_tiling_on_sc=False),
    )
    def scatter(sidx_hbm, msg_hbm, zeros_hbm, hlu_out, hli_out,
                idxv, msgv, acc):
        d = lax.axis_index("c")
        sid = lax.axis_index("s")
        pltpu.sync_copy(zeros_hbm.at[pl.ds(sid * zr, zr)],
                        acc.at[pl.ds(sid * zr, zr)])
        plsc.subcore_barrier()
        base = sid * (epad // 16)

        def body(r, carry):
            off = base + r * CH
            pltpu.sync_copy(sidx_hbm.at[d, pl.ds(off, CH)], idxv)
            pltpu.sync_copy(msg_hbm.at[d, pl.ds(off, CH)], msgv)
            pltpu.sync_copy(msgv, acc.at[idxv], add=True)
            return carry

        lax.fori_loop(0, rows, body, 0)
        plsc.subcore_barrier()

        @pl.when(d == 0)
        def _():
            oru = n_u // 16
            pltpu.sync_copy(acc.at[pl.ds(sid * oru, oru)],
                            hlu_out.at[pl.ds(sid * oru, oru)])

        @pl.when(d == 1)
        def _():
            ori = n_i // 16
            pltpu.sync_copy(acc.at[pl.ds(sid * ori, ori)],
                            hli_out.at[pl.ds(sid * ori, ori)])

    return scatter


def _tc_body(relp_ref, embp_ref, k16_ref, w1_ref, kw2_ref, kg_ref, kr_ref,
             ks_ref, out_ref):
    trep = jnp.dot(relp_ref[0], k16_ref[...],
                   preferred_element_type=jnp.float32,
                   precision=lax.Precision.HIGHEST)
    x1p = jnp.sin(trep * w1_ref[0])
    x2p = jnp.sin(jnp.dot(x1p, kw2_ref[0],
                          preferred_element_type=jnp.float32))
    xrp = jnp.dot(x2p, kr_ref[...],
                  preferred_element_type=jnp.float32)
    t2p = jnp.dot(embp_ref[0], kg_ref[0],
                  preferred_element_type=jnp.float32)
    out_ref[0] = jnp.dot(t2p * xrp, ks_ref[...],
                         preferred_element_type=jnp.float32)


def _regroup_w3(w3):
    return jnp.transpose(w3.reshape(H, H, H), (2, 0, 1)).reshape(H, H * H)


def kernel(u_embedded, i_embedded, user_per_trans, item_per_trans, edges_t,
           u_t, i_t, Wu1, Wu2, Wu3, Wi1, Wi2, Wi3):
    e = edges_t.shape[0]
    n_u = u_embedded.shape[0]
    n_i = i_embedded.shape[0]
    epad = ((e + NW * CH - 1) // (NW * CH)) * (NW * CH)
    pad = epad - e
    acc_rows = ((max(n_u, n_i) + 1 + 127) // 128) * 128

    ip = item_per_trans.astype(jnp.int32)
    up = user_per_trans.astype(jnp.int32)
    ip_g = jnp.pad(ip, (0, pad))
    up_g = jnp.pad(up, (0, pad))
    et_g = jnp.pad(edges_t, (0, pad))

    embs, rel = _gather_kernel(epad)(
        ip_g, up_g, i_embedded, u_embedded, i_t, u_t, et_g)

    te = 4096
    nb = epad // te
    eye8 = jnp.eye(8, dtype=jnp.float32)
    k16 = jnp.kron(eye8, jnp.ones((1, H), jnp.float32))
    w1s = jnp.stack([jnp.tile(OMEGA * Wi1, (1, 8)),
                     jnp.tile(OMEGA * Wu1, (1, 8))])
    kw2 = jnp.stack([jnp.kron(eye8, OMEGA * Wi2),
                     jnp.kron(eye8, OMEGA * Wu2)])
    kg = jnp.stack([jnp.kron(eye8, _regroup_w3(Wi3)),
                    jnp.kron(eye8, _regroup_w3(Wu3))])
    rm = np.kron(np.eye(H, dtype=np.float32), np.ones((1, H), np.float32))
    sm = np.kron(np.ones((H, 1), np.float32), np.eye(H, dtype=np.float32))
    kr = jnp.asarray(np.kron(np.eye(8, dtype=np.float32), rm))
    ks = jnp.asarray(np.kron(np.eye(8, dtype=np.float32), sm))

    e8 = epad // 8
    msgp = pl.pallas_call(
        _tc_body,
        grid=(2, nb),
        in_specs=[
            pl.BlockSpec((1, te // 8, 8), lambda d, b: (d, b, 0)),
            pl.BlockSpec((1, te * H // 128, 128), lambda d, b: (d, b, 0)),
            pl.BlockSpec((8, 128), lambda d, b: (0, 0)),
            pl.BlockSpec((1, 1, 128), lambda d, b: (d, 0, 0)),
            pl.BlockSpec((1, 128, 128), lambda d, b: (d, 0, 0)),
            pl.BlockSpec((1, 128, H * 128), lambda d, b: (d, 0, 0)),
            pl.BlockSpec((128, H * 128), lambda d, b: (0, 0)),
            pl.BlockSpec((H * 128, 128), lambda d, b: (0, 0)),
        ],
        out_specs=pl.BlockSpec((1, te * H // 128, 128), lambda d, b: (d, b, 0)),
        out_shape=jax.ShapeDtypeStruct((2, epad * H // 128, 128), jnp.float32),
    )(rel.reshape(2, e8, 8), embs.reshape(2, epad * H // 128, 128),
      k16, w1s, kw2, kg, kr, ks)
    msg = msgp.reshape(2, epad, H)

    sidx = jnp.stack([
        jnp.pad(up, (0, pad), constant_values=n_u),
        jnp.pad(ip, (0, pad), constant_values=n_i),
    ])
    zeros = jnp.zeros((acc_rows, H), jnp.float32)
    hlu, hli = _scatter_kernel(epad, n_u, n_i, acc_rows)(sidx, msg, zeros)
    return hlu, hli


# speedup vs baseline: 1.1973x; 1.1973x over previous
import functools

import jax
import jax.numpy as jnp
import numpy as np
from jax import lax
from jax.experimental import pallas as pl
from jax.experimental.pallas import tpu as pltpu
from jax.experimental.pallas import tpu_sc as plsc

H = 16
OMEGA = 30.0
CH = 128
NW = 32


def _gather_kernel(epad):
    rows = epad // NW // CH
    mesh = plsc.VectorSubcoreMesh(core_axis_name="c", subcore_axis_name="s")

    @functools.partial(
        pl.kernel,
        mesh=mesh,
        out_type=(
            jax.ShapeDtypeStruct((2, epad, H), jnp.float32),
            jax.ShapeDtypeStruct((2, epad), jnp.float32),
        ),
        scratch_types=[
            pltpu.VMEM((CH,), jnp.int32),
            pltpu.VMEM((CH,), jnp.int32),
            pltpu.VMEM((CH, H), jnp.float32),
            pltpu.VMEM((CH, H), jnp.float32),
            pltpu.VMEM((CH,), jnp.float32),
            pltpu.VMEM((CH,), jnp.float32),
            pltpu.VMEM((CH,), jnp.float32),
            pltpu.SemaphoreType.DMA,
        ],
        compiler_params=pltpu.CompilerParams(use_tc_tiling_on_sc=False),
    )
    def gather(ip_hbm, up_hbm, iemb_hbm, uemb_hbm, it_hbm, ut_hbm, et_hbm,
               emb_out, rel_out,
               ipv, upv, iembv, uembv, itv, utv, etv, sem):
        wid = lax.axis_index("s") * 2 + lax.axis_index("c")
        base = wid * (epad // NW)

        def body(r, carry):
            off = base + r * CH
            pltpu.sync_copy(ip_hbm.at[pl.ds(off, CH)], ipv)
            pltpu.sync_copy(up_hbm.at[pl.ds(off, CH)], upv)
            pltpu.sync_copy(et_hbm.at[pl.ds(off, CH)], etv)
            c1 = pltpu.async_copy(iemb_hbm.at[ipv], iembv, sem)
            c2 = pltpu.async_copy(uemb_hbm.at[upv], uembv, sem)
            c3 = pltpu.async_copy(it_hbm.at[ipv], itv, sem)
            c4 = pltpu.async_copy(ut_hbm.at[upv], utv, sem)
            c1.wait()
            c2.wait()
            c3.wait()
            c4.wait()
            for k in range(CH // 16):
                sl = pl.ds(k * 16, 16)
                itv[sl] = itv[sl] - etv[sl]
                utv[sl] = utv[sl] - etv[sl]
            pltpu.sync_copy(iembv, emb_out.at[0, pl.ds(off, CH)])
            pltpu.sync_copy(uembv, emb_out.at[1, pl.ds(off, CH)])
            pltpu.sync_copy(itv, rel_out.at[0, pl.ds(off, CH)])
            pltpu.sync_copy(utv, rel_out.at[1, pl.ds(off, CH)])
            return carry

        lax.fori_loop(0, rows, body, 0)

    return gather


def _scatter_kernel(epad, acc_rows):
    rows = epad // 16 // CH
    zr = acc_rows // 16
    mesh = plsc.VectorSubcoreMesh(core_axis_name="c", subcore_axis_name="s")

    @functools.partial(
        pl.kernel,
        mesh=mesh,
        out_type=jax.ShapeDtypeStruct((2, acc_rows, H), jnp.float32),
        scratch_types=[
            pltpu.VMEM((CH,), jnp.int32),
            pltpu.VMEM((CH, H), jnp.float32),
            pltpu.VMEM_SHARED((acc_rows, H), jnp.float32),
        ],
        compiler_params=pltpu.CompilerParams(use_tc_tiling_on_sc=False),
    )
    def scatter(sidx_hbm, msg_hbm, init_hbm, hl_out, idxv, msgv, acc):
        d = lax.axis_index("c")
        sid = lax.axis_index("s")
        pltpu.sync_copy(init_hbm.at[d, pl.ds(sid * zr, zr)],
                        acc.at[pl.ds(sid * zr, zr)])
        plsc.subcore_barrier()
        base = sid * (epad // 16)

        def body(r, carry):
            off = base + r * CH
            pltpu.sync_copy(sidx_hbm.at[d, pl.ds(off, CH)], idxv)
            pltpu.sync_copy(msg_hbm.at[d, pl.ds(off, CH)], msgv)
            pltpu.sync_copy(msgv, acc.at[idxv], add=True)
            return carry

        lax.fori_loop(0, rows, body, 0)
        plsc.subcore_barrier()
        pltpu.sync_copy(acc.at[pl.ds(sid * zr, zr)],
                        hl_out.at[d, pl.ds(sid * zr, zr)])

    return scatter


def _tc_body(relp_ref, embp_ref, k16_ref, w1_ref, kw2_ref, kg_ref, kr_ref,
             ks_ref, out_ref):
    trep = jnp.dot(relp_ref[0], k16_ref[...],
                   preferred_element_type=jnp.float32,
                   precision=lax.Precision.HIGHEST)
    x1p = jnp.sin(trep * w1_ref[0])
    x2p = jnp.sin(jnp.dot(x1p, kw2_ref[0],
                          preferred_element_type=jnp.float32))
    xrp = jnp.dot(x2p, kr_ref[...],
                  preferred_element_type=jnp.float32)
    t2p = jnp.dot(embp_ref[0], kg_ref[0],
                  preferred_element_type=jnp.float32)
    out_ref[0] = jnp.dot(t2p * xrp, ks_ref[...],
                         preferred_element_type=jnp.float32)


def _regroup_w3(w3):
    return jnp.transpose(w3.reshape(H, H, H), (2, 0, 1)).reshape(H, H * H)


def kernel(u_embedded, i_embedded, user_per_trans, item_per_trans, edges_t,
           u_t, i_t, Wu1, Wu2, Wu3, Wi1, Wi2, Wi3):
    e = edges_t.shape[0]
    n_u = u_embedded.shape[0]
    n_i = i_embedded.shape[0]
    epad = ((e + NW * CH - 1) // (NW * CH)) * (NW * CH)
    pad = epad - e
    acc_rows = ((max(n_u, n_i) + 1 + 127) // 128) * 128

    ip = item_per_trans.astype(jnp.int32)
    up = user_per_trans.astype(jnp.int32)
    ip_g = jnp.pad(ip, (0, pad))
    up_g = jnp.pad(up, (0, pad))
    et_g = jnp.pad(edges_t, (0, pad))

    te = 4096
    eye8 = jnp.eye(8, dtype=jnp.float32)
    k16 = jnp.kron(eye8, jnp.ones((1, H), jnp.float32))
    w1s = jnp.stack([jnp.tile(OMEGA * Wi1, (1, 8)),
                     jnp.tile(OMEGA * Wu1, (1, 8))])
    kw2 = jnp.stack([jnp.kron(eye8, OMEGA * Wi2),
                     jnp.kron(eye8, OMEGA * Wu2)])
    kg = jnp.stack([jnp.kron(eye8, _regroup_w3(Wi3)),
                    jnp.kron(eye8, _regroup_w3(Wu3))])
    rm = np.kron(np.eye(H, dtype=np.float32), np.ones((1, H), np.float32))
    sm = np.kron(np.ones((H, 1), np.float32), np.eye(H, dtype=np.float32))
    kr = jnp.asarray(np.kron(np.eye(8, dtype=np.float32), rm))
    ks = jnp.asarray(np.kron(np.eye(8, dtype=np.float32), sm))

    sidx = jnp.stack([
        jnp.pad(up, (0, pad), constant_values=n_u),
        jnp.pad(ip, (0, pad), constant_values=n_i),
    ])

    nchunks = 2
    ec = epad // nchunks
    nb = ec // te
    gather = _gather_kernel(ec)
    scatter = _scatter_kernel(ec, acc_rows)
    hl = jnp.zeros((2, acc_rows, H), jnp.float32)
    for c in range(nchunks):
        sl = slice(c * ec, (c + 1) * ec)
        embs, rel = gather(ip_g[sl], up_g[sl], i_embedded, u_embedded,
                           i_t, u_t, et_g[sl])
        msgp = pl.pallas_call(
            _tc_body,
            grid=(2, nb),
            in_specs=[
                pl.BlockSpec((1, te // 8, 8), lambda d, b: (d, b, 0)),
                pl.BlockSpec((1, te * H // 128, 128), lambda d, b: (d, b, 0)),
                pl.BlockSpec((8, 128), lambda d, b: (0, 0)),
                pl.BlockSpec((1, 1, 128), lambda d, b: (d, 0, 0)),
                pl.BlockSpec((1, 128, 128), lambda d, b: (d, 0, 0)),
                pl.BlockSpec((1, 128, H * 128), lambda d, b: (d, 0, 0)),
                pl.BlockSpec((128, H * 128), lambda d, b: (0, 0)),
                pl.BlockSpec((H * 128, 128), lambda d, b: (0, 0)),
            ],
            out_specs=pl.BlockSpec((1, te * H // 128, 128),
                                   lambda d, b: (d, b, 0)),
            out_shape=jax.ShapeDtypeStruct((2, ec * H // 128, 128),
                                           jnp.float32),
        )(rel.reshape(2, ec // 8, 8), embs.reshape(2, ec * H // 128, 128),
          k16, w1s, kw2, kg, kr, ks)
        msg = msgp.reshape(2, ec, H)
        hl = scatter(sidx[:, sl], msg, hl)
    return hl[0, :n_u], hl[1, :n_i]


# speedup vs baseline: 1.2995x; 1.0853x over previous
import functools

import jax
import jax.numpy as jnp
import numpy as np
from jax import lax
from jax.experimental import pallas as pl
from jax.experimental.pallas import tpu as pltpu
from jax.experimental.pallas import tpu_sc as plsc

H = 16
OMEGA = 30.0
CH = 128
NW = 32


def _gather_kernel(epad):
    rows = epad // NW // CH
    mesh = plsc.VectorSubcoreMesh(core_axis_name="c", subcore_axis_name="s")

    @functools.partial(
        pl.kernel,
        mesh=mesh,
        out_type=(
            jax.ShapeDtypeStruct((2, epad, H), jnp.float32),
            jax.ShapeDtypeStruct((2, epad), jnp.float32),
        ),
        scratch_types=[
            pltpu.VMEM((CH,), jnp.int32),
            pltpu.VMEM((CH,), jnp.int32),
            pltpu.VMEM((CH, H), jnp.float32),
            pltpu.VMEM((CH, H), jnp.float32),
            pltpu.VMEM((CH,), jnp.float32),
            pltpu.VMEM((CH,), jnp.float32),
            pltpu.VMEM((CH,), jnp.float32),
            pltpu.SemaphoreType.DMA,
        ],
        compiler_params=pltpu.CompilerParams(use_tc_tiling_on_sc=False),
    )
    def gather(ip_hbm, up_hbm, iemb_hbm, uemb_hbm, it_hbm, ut_hbm, et_hbm,
               emb_out, rel_out,
               ipv, upv, iembv, uembv, itv, utv, etv, sem):
        wid = lax.axis_index("s") * 2 + lax.axis_index("c")
        base = wid * (epad // NW)

        def body(r, carry):
            off = base + r * CH
            pltpu.sync_copy(ip_hbm.at[pl.ds(off, CH)], ipv)
            pltpu.sync_copy(up_hbm.at[pl.ds(off, CH)], upv)
            pltpu.sync_copy(et_hbm.at[pl.ds(off, CH)], etv)
            c1 = pltpu.async_copy(iemb_hbm.at[ipv], iembv, sem)
            c2 = pltpu.async_copy(uemb_hbm.at[upv], uembv, sem)
            c3 = pltpu.async_copy(it_hbm.at[ipv], itv, sem)
            c4 = pltpu.async_copy(ut_hbm.at[upv], utv, sem)
            c1.wait()
            c2.wait()
            c3.wait()
            c4.wait()
            for k in range(CH // 16):
                sl = pl.ds(k * 16, 16)
                itv[sl] = itv[sl] - etv[sl]
                utv[sl] = utv[sl] - etv[sl]
            pltpu.sync_copy(iembv, emb_out.at[0, pl.ds(off, CH)])
            pltpu.sync_copy(uembv, emb_out.at[1, pl.ds(off, CH)])
            pltpu.sync_copy(itv, rel_out.at[0, pl.ds(off, CH)])
            pltpu.sync_copy(utv, rel_out.at[1, pl.ds(off, CH)])
            return carry

        lax.fori_loop(0, rows, body, 0)

    return gather


def _scatter_kernel(epad, acc_rows):
    rows = epad // 16 // CH
    zr = acc_rows // 16
    mesh = plsc.VectorSubcoreMesh(core_axis_name="c", subcore_axis_name="s")

    @functools.partial(
        pl.kernel,
        mesh=mesh,
        out_type=jax.ShapeDtypeStruct((2, acc_rows, H), jnp.float32),
        scratch_types=[
            pltpu.VMEM((CH,), jnp.int32),
            pltpu.VMEM((CH, H), jnp.float32),
            pltpu.VMEM_SHARED((acc_rows, H), jnp.float32),
        ],
        compiler_params=pltpu.CompilerParams(use_tc_tiling_on_sc=False),
    )
    def scatter(sidx_hbm, msg_hbm, init_hbm, hl_out, idxv, msgv, acc):
        d = lax.axis_index("c")
        sid = lax.axis_index("s")
        pltpu.sync_copy(init_hbm.at[d, pl.ds(sid * zr, zr)],
                        acc.at[pl.ds(sid * zr, zr)])
        plsc.subcore_barrier()
        base = sid * (epad // 16)

        def body(r, carry):
            off = base + r * CH
            pltpu.sync_copy(sidx_hbm.at[d, pl.ds(off, CH)], idxv)
            pltpu.sync_copy(msg_hbm.at[d, pl.ds(off, CH)], msgv)
            pltpu.sync_copy(msgv, acc.at[idxv], add=True)
            return carry

        lax.fori_loop(0, rows, body, 0)
        plsc.subcore_barrier()
        pltpu.sync_copy(acc.at[pl.ds(sid * zr, zr)],
                        hl_out.at[d, pl.ds(sid * zr, zr)])

    return scatter


def _tc_body(relp_ref, embp_ref, k16_ref, w1_ref, kw2_ref, kg_ref, kr_ref,
             ks_ref, out_ref):
    trep = jnp.dot(relp_ref[0], k16_ref[...],
                   preferred_element_type=jnp.float32,
                   precision=lax.Precision.HIGHEST)
    x1p = jnp.sin(trep * w1_ref[0])
    x2p = jnp.sin(jnp.dot(x1p, kw2_ref[0],
                          preferred_element_type=jnp.float32))
    xrp = jnp.dot(x2p, kr_ref[...],
                  preferred_element_type=jnp.float32)
    t2p = jnp.dot(embp_ref[0], kg_ref[0],
                  preferred_element_type=jnp.float32)
    out_ref[0] = jnp.dot(t2p * xrp, ks_ref[...],
                         preferred_element_type=jnp.float32)


def _regroup_w3(w3):
    return jnp.transpose(w3.reshape(H, H, H), (2, 0, 1)).reshape(H, H * H)


def kernel(u_embedded, i_embedded, user_per_trans, item_per_trans, edges_t,
           u_t, i_t, Wu1, Wu2, Wu3, Wi1, Wi2, Wi3):
    e = edges_t.shape[0]
    n_u = u_embedded.shape[0]
    n_i = i_embedded.shape[0]
    epad = ((e + NW * CH - 1) // (NW * CH)) * (NW * CH)
    pad = epad - e
    acc_rows = ((max(n_u, n_i) + 1 + 127) // 128) * 128

    ip = item_per_trans.astype(jnp.int32)
    up = user_per_trans.astype(jnp.int32)
    ip_g = jnp.pad(ip, (0, pad))
    up_g = jnp.pad(up, (0, pad))
    et_g = jnp.pad(edges_t, (0, pad))

    te = 4096
    eye8 = jnp.eye(8, dtype=jnp.float32)
    k16 = jnp.kron(eye8, jnp.ones((1, H), jnp.float32))
    w1s = jnp.stack([jnp.tile(OMEGA * Wi1, (1, 8)),
                     jnp.tile(OMEGA * Wu1, (1, 8))])
    kw2 = jnp.stack([jnp.kron(eye8, OMEGA * Wi2),
                     jnp.kron(eye8, OMEGA * Wu2)])
    kg = jnp.stack([jnp.kron(eye8, _regroup_w3(Wi3)),
                    jnp.kron(eye8, _regroup_w3(Wu3))])
    rm = np.kron(np.eye(H, dtype=np.float32), np.ones((1, H), np.float32))
    sm = np.kron(np.ones((H, 1), np.float32), np.eye(H, dtype=np.float32))
    kr = jnp.asarray(np.kron(np.eye(8, dtype=np.float32), rm))
    ks = jnp.asarray(np.kron(np.eye(8, dtype=np.float32), sm))

    sidx = jnp.stack([
        jnp.pad(up, (0, pad), constant_values=n_u),
        jnp.pad(ip, (0, pad), constant_values=n_i),
    ])

    nchunks = 4
    ec = epad // nchunks
    nb = ec // te
    gather = _gather_kernel(ec)
    scatter = _scatter_kernel(ec, acc_rows)
    hl = jnp.zeros((2, acc_rows, H), jnp.float32)
    for c in range(nchunks):
        sl = slice(c * ec, (c + 1) * ec)
        embs, rel = gather(ip_g[sl], up_g[sl], i_embedded, u_embedded,
                           i_t, u_t, et_g[sl])
        msgp = pl.pallas_call(
            _tc_body,
            grid=(2, nb),
            in_specs=[
                pl.BlockSpec((1, te // 8, 8), lambda d, b: (d, b, 0)),
                pl.BlockSpec((1, te * H // 128, 128), lambda d, b: (d, b, 0)),
                pl.BlockSpec((8, 128), lambda d, b: (0, 0)),
                pl.BlockSpec((1, 1, 128), lambda d, b: (d, 0, 0)),
                pl.BlockSpec((1, 128, 128), lambda d, b: (d, 0, 0)),
                pl.BlockSpec((1, 128, H * 128), lambda d, b: (d, 0, 0)),
                pl.BlockSpec((128, H * 128), lambda d, b: (0, 0)),
                pl.BlockSpec((H * 128, 128), lambda d, b: (0, 0)),
            ],
            out_specs=pl.BlockSpec((1, te * H // 128, 128),
                                   lambda d, b: (d, b, 0)),
            out_shape=jax.ShapeDtypeStruct((2, ec * H // 128, 128),
                                           jnp.float32),
        )(rel.reshape(2, ec // 8, 8), embs.reshape(2, ec * H // 128, 128),
          k16, w1s, kw2, kg, kr, ks)
        msg = msgp.reshape(2, ec, H)
        hl = scatter(sidx[:, sl], msg, hl)
    return hl[0, :n_u], hl[1, :n_i]


# speedup vs baseline: 1.3120x; 1.0097x over previous
import functools

import jax
import jax.numpy as jnp
import numpy as np
from jax import lax
from jax.experimental import pallas as pl
from jax.experimental.pallas import tpu as pltpu
from jax.experimental.pallas import tpu_sc as plsc

H = 16
OMEGA = 30.0
CH = 128
NW = 32


def _gather_kernel(epad):
    rows = epad // NW // CH
    mesh = plsc.VectorSubcoreMesh(core_axis_name="c", subcore_axis_name="s")

    @functools.partial(
        pl.kernel,
        mesh=mesh,
        out_type=(
            jax.ShapeDtypeStruct((2, epad, H), jnp.float32),
            jax.ShapeDtypeStruct((2, epad), jnp.float32),
        ),
        scratch_types=[
            pltpu.VMEM((CH,), jnp.int32),
            pltpu.VMEM((CH,), jnp.int32),
            pltpu.VMEM((CH, H), jnp.float32),
            pltpu.VMEM((CH, H), jnp.float32),
            pltpu.VMEM((CH,), jnp.float32),
            pltpu.VMEM((CH,), jnp.float32),
            pltpu.VMEM((CH,), jnp.float32),
            pltpu.SemaphoreType.DMA,
        ],
        compiler_params=pltpu.CompilerParams(use_tc_tiling_on_sc=False),
    )
    def gather(ip_hbm, up_hbm, iemb_hbm, uemb_hbm, it_hbm, ut_hbm, et_hbm,
               emb_out, rel_out,
               ipv, upv, iembv, uembv, itv, utv, etv, sem):
        wid = lax.axis_index("s") * 2 + lax.axis_index("c")
        base = wid * (epad // NW)

        def body(r, carry):
            off = base + r * CH
            pltpu.sync_copy(ip_hbm.at[pl.ds(off, CH)], ipv)
            pltpu.sync_copy(up_hbm.at[pl.ds(off, CH)], upv)
            pltpu.sync_copy(et_hbm.at[pl.ds(off, CH)], etv)
            c1 = pltpu.async_copy(iemb_hbm.at[ipv], iembv, sem)
            c2 = pltpu.async_copy(uemb_hbm.at[upv], uembv, sem)
            c3 = pltpu.async_copy(it_hbm.at[ipv], itv, sem)
            c4 = pltpu.async_copy(ut_hbm.at[upv], utv, sem)
            c1.wait()
            c2.wait()
            c3.wait()
            c4.wait()
            for k in range(CH // 16):
                sl = pl.ds(k * 16, 16)
                itv[sl] = itv[sl] - etv[sl]
                utv[sl] = utv[sl] - etv[sl]
            pltpu.sync_copy(iembv, emb_out.at[0, pl.ds(off, CH)])
            pltpu.sync_copy(uembv, emb_out.at[1, pl.ds(off, CH)])
            pltpu.sync_copy(itv, rel_out.at[0, pl.ds(off, CH)])
            pltpu.sync_copy(utv, rel_out.at[1, pl.ds(off, CH)])
            return carry

        lax.fori_loop(0, rows, body, 0)

    return gather


def _scatter_kernel(epad, acc_rows):
    rows = epad // 16 // CH
    zr = acc_rows // 16
    mesh = plsc.VectorSubcoreMesh(core_axis_name="c", subcore_axis_name="s")

    @functools.partial(
        pl.kernel,
        mesh=mesh,
        out_type=jax.ShapeDtypeStruct((2, acc_rows, H), jnp.float32),
        scratch_types=[
            pltpu.VMEM((CH,), jnp.int32),
            pltpu.VMEM((CH, H), jnp.float32),
            pltpu.VMEM_SHARED((acc_rows, H), jnp.float32),
        ],
        compiler_params=pltpu.CompilerParams(use_tc_tiling_on_sc=False),
    )
    def scatter(sidx_hbm, msg_hbm, init_hbm, hl_out, idxv, msgv, acc):
        d = lax.axis_index("c")
        sid = lax.axis_index("s")
        pltpu.sync_copy(init_hbm.at[d, pl.ds(sid * zr, zr)],
                        acc.at[pl.ds(sid * zr, zr)])
        plsc.subcore_barrier()
        base = sid * (epad // 16)

        def body(r, carry):
            off = base + r * CH
            pltpu.sync_copy(sidx_hbm.at[d, pl.ds(off, CH)], idxv)
            pltpu.sync_copy(msg_hbm.at[d, pl.ds(off, CH)], msgv)
            pltpu.sync_copy(msgv, acc.at[idxv], add=True)
            return carry

        lax.fori_loop(0, rows, body, 0)
        plsc.subcore_barrier()
        pltpu.sync_copy(acc.at[pl.ds(sid * zr, zr)],
                        hl_out.at[d, pl.ds(sid * zr, zr)])

    return scatter


def _tc_body(relp_ref, embp_ref, k16_ref, w1_ref, kw2_ref, kg_ref, kr_ref,
             ks_ref, out_ref):
    trep = jnp.dot(relp_ref[0], k16_ref[...],
                   preferred_element_type=jnp.float32,
                   precision=lax.Precision.HIGHEST)
    x1p = jnp.sin(trep * w1_ref[0])
    x2p = jnp.sin(jnp.dot(x1p, kw2_ref[0],
                          preferred_element_type=jnp.float32))
    xrp = jnp.dot(x2p, kr_ref[...],
                  preferred_element_type=jnp.float32)
    t2p = jnp.dot(embp_ref[0], kg_ref[0],
                  preferred_element_type=jnp.float32)
    out_ref[0] = jnp.dot(t2p * xrp, ks_ref[...],
                         preferred_element_type=jnp.float32)


def _regroup_w3(w3):
    return jnp.transpose(w3.reshape(H, H, H), (2, 0, 1)).reshape(H, H * H)


def kernel(u_embedded, i_embedded, user_per_trans, item_per_trans, edges_t,
           u_t, i_t, Wu1, Wu2, Wu3, Wi1, Wi2, Wi3):
    e = edges_t.shape[0]
    n_u = u_embedded.shape[0]
    n_i = i_embedded.shape[0]
    epad = ((e + NW * CH - 1) // (NW * CH)) * (NW * CH)
    pad = epad - e
    acc_rows = ((max(n_u, n_i) + 1 + 127) // 128) * 128

    ip = item_per_trans.astype(jnp.int32)
    up = user_per_trans.astype(jnp.int32)
    ip_g = jnp.pad(ip, (0, pad))
    up_g = jnp.pad(up, (0, pad))
    et_g = jnp.pad(edges_t, (0, pad))

    te = 4096
    eye8 = jnp.eye(8, dtype=jnp.float32)
    k16 = jnp.kron(eye8, jnp.ones((1, H), jnp.float32))
    w1s = jnp.stack([jnp.tile(OMEGA * Wi1, (1, 8)),
                     jnp.tile(OMEGA * Wu1, (1, 8))])
    kw2 = jnp.stack([jnp.kron(eye8, OMEGA * Wi2),
                     jnp.kron(eye8, OMEGA * Wu2)])
    kg = jnp.stack([jnp.kron(eye8, _regroup_w3(Wi3)),
                    jnp.kron(eye8, _regroup_w3(Wu3))])
    rm = np.kron(np.eye(H, dtype=np.float32), np.ones((1, H), np.float32))
    sm = np.kron(np.ones((H, 1), np.float32), np.eye(H, dtype=np.float32))
    kr = jnp.asarray(np.kron(np.eye(8, dtype=np.float32), rm))
    ks = jnp.asarray(np.kron(np.eye(8, dtype=np.float32), sm))

    sidx = jnp.stack([
        jnp.pad(up, (0, pad), constant_values=n_u),
        jnp.pad(ip, (0, pad), constant_values=n_i),
    ])

    nchunks = 8
    ec = epad // nchunks
    nb = ec // te
    gather = _gather_kernel(ec)
    scatter = _scatter_kernel(ec, acc_rows)
    hl = jnp.zeros((2, acc_rows, H), jnp.float32)
    for c in range(nchunks):
        sl = slice(c * ec, (c + 1) * ec)
        embs, rel = gather(ip_g[sl], up_g[sl], i_embedded, u_embedded,
                           i_t, u_t, et_g[sl])
        msgp = pl.pallas_call(
            _tc_body,
            grid=(2, nb),
            in_specs=[
                pl.BlockSpec((1, te // 8, 8), lambda d, b: (d, b, 0)),
                pl.BlockSpec((1, te * H // 128, 128), lambda d, b: (d, b, 0)),
                pl.BlockSpec((8, 128), lambda d, b: (0, 0)),
                pl.BlockSpec((1, 1, 128), lambda d, b: (d, 0, 0)),
                pl.BlockSpec((1, 128, 128), lambda d, b: (d, 0, 0)),
                pl.BlockSpec((1, 128, H * 128), lambda d, b: (d, 0, 0)),
                pl.BlockSpec((128, H * 128), lambda d, b: (0, 0)),
                pl.BlockSpec((H * 128, 128), lambda d, b: (0, 0)),
            ],
            out_specs=pl.BlockSpec((1, te * H // 128, 128),
                                   lambda d, b: (d, b, 0)),
            out_shape=jax.ShapeDtypeStruct((2, ec * H // 128, 128),
                                           jnp.float32),
        )(rel.reshape(2, ec // 8, 8), embs.reshape(2, ec * H // 128, 128),
          k16, w1s, kw2, kg, kr, ks)
        msg = msgp.reshape(2, ec, H)
        hl = scatter(sidx[:, sl], msg, hl)
    return hl[0, :n_u], hl[1, :n_i]
